# baseline (speedup 1.0000x reference)
import functools

import jax
import jax.numpy as jnp
from jax import lax
from jax.experimental import pallas as pl
from jax.experimental.pallas import tpu as pltpu
from jax.experimental.pallas import tpu_sc as plsc

N = 100000
E = 6400000
NPAD = 102400
EP = 6553600
EW = EP // 32
K = 2048
NCHUNK = EW // K
KR = K // 128
GU = 8
NSL = NPAD // 16

_f32 = jnp.float32
_i32 = jnp.int32


def _m8(v):
    return pl.multiple_of(v, 8)


def _mesh():
    return plsc.VectorSubcoreMesh(core_axis_name="c", subcore_axis_name="s")


def _rsqrt16(x):
    i = lax.bitcast_convert_type(x, _i32)
    i = jnp.full((16,), 0x5F3759DF, _i32) - lax.shift_right_logical(i, 1)
    y = lax.bitcast_convert_type(i, _f32)
    half = x * jnp.full((16,), 0.5, _f32)
    c15 = jnp.full((16,), 1.5, _f32)
    for _ in range(3):
        y = y * (c15 - half * y * y)
    return y


def _make_prop(D, gather):
    scratch = [
        pltpu.VMEM((NPAD,), _f32),
        pltpu.VMEM((K,), _i32),
        pltpu.VMEM((K,), _i32),
        pltpu.VMEM((KR, 128), _i32),
        pltpu.VMEM((KR, 128), _i32),
        pltpu.VMEM((K,), _f32),
        pltpu.VMEM((K,), _f32),
        pltpu.VMEM_SHARED((NPAD,), _f32),
        pltpu.SemaphoreType.DMA,
        pltpu.SemaphoreType.DMA,
    ]

    @functools.partial(
        pl.kernel,
        out_type=jax.ShapeDtypeStruct((2 * D * NPAD,), _f32),
        mesh=_mesh(),
        compiler_params=pltpu.CompilerParams(needs_layout_passes=False),
        scratch_types=scratch,
    )
    def prop(u_h, src_h, dst_h, z_h, out_h, tab_v, src_v0, src_v1,
             dst_v0, dst_v1, val_v0, val_v1, acc, sem, sem2):
        c = lax.axis_index("c")
        s = lax.axis_index("s")
        w = c * 16 + s
        ebase = _m8(w * EW)
        rbase = _m8(w * (EW // 128))
        src_b = (src_v0, src_v1)
        dst_b = (dst_v0, dst_v1)
        val_b = (val_v0, val_v1)

        if not gather:
            def fill(i, carry):
                val_v0[pl.ds(i * 16, 16)] = jnp.full((16,), 1.0, _f32)
                val_v1[pl.ds(i * 16, 16)] = jnp.full((16,), 1.0, _f32)
                return carry
            lax.fori_loop(0, K // 16, fill, 0)

        def start_idx_load(i, b):
            pltpu.async_copy(dst_h.at[pl.ds(_m8(rbase + i * KR), KR)],
                             dst_b[b], sem2)
            if gather:
                pltpu.async_copy(src_h.at[pl.ds(_m8(ebase + i * K), K)],
                                 src_b[b], sem2)

        def wait_idx_load(b):
            pltpu.make_async_copy(dst_h.at[pl.ds(0, KR)], dst_b[b], sem2).wait()
            if gather:
                pltpu.make_async_copy(src_h.at[pl.ds(0, K)], src_b[b],
                                      sem2).wait()

        def gather_fire_chunk(b):
            def gat(k, carry3):
                kb = k * 128
                if gather:
                    for g in range(GU):
                        idx = src_b[b][pl.ds(kb + g * 16, 16)]
                        val_b[b][pl.ds(kb + g * 16, 16)] = (
                            plsc.load_gather(tab_v, [idx]))
                pltpu.async_copy(val_b[b].at[pl.ds(kb, 128)],
                                 acc.at[dst_b[b].at[k]], sem, add=True)
                return carry3
            lax.fori_loop(0, KR, gat, 0)

        def feature(j, carry):
            sl = _m8(s * NSL)
            pltpu.sync_copy(z_h.at[pl.ds(sl, NSL)], acc.at[pl.ds(sl, NSL)])
            if gather:
                pltpu.sync_copy(u_h.at[pl.ds(_m8(j * NPAD), NPAD)], tab_v)
            plsc.subcore_barrier()

            start_idx_load(0, 0)

            def chunk2(ih, carry2):
                for b in range(2):
                    i = ih * 2 + b
                    wait_idx_load(b)
                    gather_fire_chunk(b)

                    @pl.when(i > 0)
                    def _():
                        pltpu.make_async_copy(z_h.at[pl.ds(0, K)],
                                              val_b[1 - b], sem).wait()

                    @pl.when(i + 1 < NCHUNK)
                    def _():
                        start_idx_load(i + 1, 1 - b)
                return carry2
            lax.fori_loop(0, NCHUNK // 2, chunk2, 0)
            pltpu.make_async_copy(z_h.at[pl.ds(0, K)],
                                  val_b[(NCHUNK - 1) % 2], sem).wait()

            plsc.subcore_barrier()
            pltpu.sync_copy(acc.at[pl.ds(sl, NSL)],
                            out_h.at[pl.ds(_m8((c * D + j) * NPAD + sl), NSL)])
            plsc.subcore_barrier()
            return carry
        lax.fori_loop(0, D, feature, 0)

    return prop


def _worker_slice():
    c = lax.axis_index("c")
    s = lax.axis_index("s")
    return _m8((c * 16 + s) * 3200)


@functools.partial(
    pl.kernel,
    out_type=(
        jax.ShapeDtypeStruct((3 * NPAD,), _f32),
        jax.ShapeDtypeStruct((3 * NPAD,), _f32),
        jax.ShapeDtypeStruct((NPAD,), _f32),
        jax.ShapeDtypeStruct((NPAD,), _f32),
    ),
    mesh=_mesh(),
    compiler_params=pltpu.CompilerParams(needs_layout_passes=False),
    scratch_types=[
        pltpu.VMEM((2 * 3200,), _f32),
        pltpu.VMEM((3 * 3200,), _f32),
        pltpu.VMEM((3 * 3200,), _f32),
        pltpu.VMEM((3 * 3200,), _f32),
        pltpu.VMEM((3200,), _f32),
        pltpu.VMEM((3200,), _f32),
    ],
)
def _prep0(dp_h, xt_h, u1_h, t1_h, di_h, di2_h,
           dp_v, x_v, u_v, t_v, di_v, di2_v):
    nb = _worker_slice()
    for k in range(2):
        pltpu.sync_copy(dp_h.at[pl.ds(_m8(k * NPAD + nb), 3200)],
                        dp_v.at[pl.ds(k * 3200, 3200)])
    for k in range(3):
        pltpu.sync_copy(xt_h.at[pl.ds(_m8(k * NPAD + nb), 3200)],
                        x_v.at[pl.ds(k * 3200, 3200)])

    one = jnp.full((16,), 1.0, _f32)

    def pos(p, carry):
        o = p * 16
        d = dp_v[pl.ds(o, 16)] + dp_v[pl.ds(3200 + o, 16)] + one
        y = _rsqrt16(d)
        y2 = y * y
        di_v[pl.ds(o, 16)] = y
        di2_v[pl.ds(o, 16)] = y2
        for k in range(3):
            xv = x_v[pl.ds(k * 3200 + o, 16)]
            u_v[pl.ds(k * 3200 + o, 16)] = xv * y
            t_v[pl.ds(k * 3200 + o, 16)] = xv * y2
        return carry
    lax.fori_loop(0, 200, pos, 0)

    for k in range(3):
        pltpu.sync_copy(u_v.at[pl.ds(k * 3200, 3200)],
                        u1_h.at[pl.ds(_m8(k * NPAD + nb), 3200)])
        pltpu.sync_copy(t_v.at[pl.ds(k * 3200, 3200)],
                        t1_h.at[pl.ds(_m8(k * NPAD + nb), 3200)])
    pltpu.sync_copy(di_v, di_h.at[pl.ds(nb, 3200)])
    pltpu.sync_copy(di2_v, di2_h.at[pl.ds(nb, 3200)])


@functools.partial(
    pl.kernel,
    out_type=(
        jax.ShapeDtypeStruct((16 * NPAD,), _f32),
        jax.ShapeDtypeStruct((16 * NPAD,), _f32),
    ),
    mesh=_mesh(),
    compiler_params=pltpu.CompilerParams(needs_layout_passes=False),
    scratch_types=[
        pltpu.VMEM((6 * 1600,), _f32),
        pltpu.VMEM((3 * 1600,), _f32),
        pltpu.VMEM((1600,), _f32),
        pltpu.VMEM((1600,), _f32),
        pltpu.VMEM((48 * 16,), _f32),
        pltpu.VMEM((16 * 16,), _f32),
        pltpu.VMEM((16 * 1600,), _f32),
        pltpu.VMEM((16 * 1600,), _f32),
    ],
)
def _mix1(p_h, t_h, di_h, di2_h, w_h, b_h, u2_h, t2_h,
          p_v, t_v, di_v, di2_v, w_v, b_v, u_v, tv_v):
    nb = _worker_slice()
    pltpu.sync_copy(w_h, w_v)
    pltpu.sync_copy(b_h, b_v)

    def piece(pc, carry):
        pb = _m8(nb + pc * 1600)
        for k in range(3):
            pltpu.sync_copy(p_h.at[pl.ds(_m8(k * NPAD + pb), 1600)],
                            p_v.at[pl.ds(k * 1600, 1600)])
            pltpu.sync_copy(p_h.at[pl.ds(_m8((3 + k) * NPAD + pb), 1600)],
                            p_v.at[pl.ds((3 + k) * 1600, 1600)])
            pltpu.sync_copy(t_h.at[pl.ds(_m8(k * NPAD + pb), 1600)],
                            t_v.at[pl.ds(k * 1600, 1600)])
        pltpu.sync_copy(di_h.at[pl.ds(pb, 1600)], di_v)
        pltpu.sync_copy(di2_h.at[pl.ds(pb, 1600)], di2_v)

        def pos(p, carry2):
            o = p * 16
            y = di_v[pl.ds(o, 16)]
            y2 = di2_v[pl.ds(o, 16)]
            zero = jnp.full((16,), 0.0, _f32)
            pk = []
            for k in range(3):
                pk.append(
                    (p_v[pl.ds(k * 1600 + o, 16)]
                     + p_v[pl.ds((3 + k) * 1600 + o, 16)]) * y
                    + t_v[pl.ds(k * 1600 + o, 16)])
            for j in range(16):
                acc = b_v[pl.ds(j * 16, 16)]
                for k in range(3):
                    acc = acc + pk[k] * w_v[pl.ds((k * 16 + j) * 16, 16)]
                h = jnp.maximum(acc, zero)
                u_v[pl.ds(j * 1600 + o, 16)] = h * y
                tv_v[pl.ds(j * 1600 + o, 16)] = h * y2
            return carry2
        lax.fori_loop(0, 100, pos, 0)

        for j in range(16):
            pltpu.sync_copy(u_v.at[pl.ds(j * 1600, 1600)],
                            u2_h.at[pl.ds(_m8(j * NPAD + pb), 1600)])
            pltpu.sync_copy(tv_v.at[pl.ds(j * 1600, 1600)],
                            t2_h.at[pl.ds(_m8(j * NPAD + pb), 1600)])
        return carry
    lax.fori_loop(0, 2, piece, 0)


@functools.partial(
    pl.kernel,
    out_type=(
        jax.ShapeDtypeStruct((2 * NPAD,), _f32),
        jax.ShapeDtypeStruct((2 * NPAD,), _f32),
    ),
    mesh=_mesh(),
    compiler_params=pltpu.CompilerParams(needs_layout_passes=False),
    scratch_types=[
        pltpu.VMEM((32 * 800,), _f32),
        pltpu.VMEM((16 * 800,), _f32),
        pltpu.VMEM((800,), _f32),
        pltpu.VMEM((800,), _f32),
        pltpu.VMEM((512 * 16,), _f32),
        pltpu.VMEM((32 * 16,), _f32),
        pltpu.VMEM((64 * 16,), _f32),
        pltpu.VMEM((16 * 800,), _f32),
        pltpu.VMEM((32 * 800,), _f32),
        pltpu.VMEM((2 * 800,), _f32),
        pltpu.VMEM((2 * 800,), _f32),
    ],
)
def _mix2(p_h, t_h, di_h, di2_h, w2_h, b2_h, w3_h, u3_h, t3_h,
          p_v, t_v, di_v, di2_v, w2_v, b2_v, w3_v, q_v, h_v, u_v, tv_v):
    nb = _worker_slice()
    pltpu.sync_copy(w2_h, w2_v)
    pltpu.sync_copy(b2_h, b2_v)
    pltpu.sync_copy(w3_h, w3_v)

    def piece(pc, carry):
        pb = _m8(nb + pc * 800)
        for j in range(16):
            pltpu.sync_copy(p_h.at[pl.ds(_m8(j * NPAD + pb), 800)],
                            p_v.at[pl.ds(j * 800, 800)])
            pltpu.sync_copy(p_h.at[pl.ds(_m8((16 + j) * NPAD + pb), 800)],
                            p_v.at[pl.ds((16 + j) * 800, 800)])
            pltpu.sync_copy(t_h.at[pl.ds(_m8(j * NPAD + pb), 800)],
                            t_v.at[pl.ds(j * 800, 800)])
        pltpu.sync_copy(di_h.at[pl.ds(pb, 800)], di_v)
        pltpu.sync_copy(di2_h.at[pl.ds(pb, 800)], di2_v)

        def posq(p, carry2):
            o = p * 16
            y = di_v[pl.ds(o, 16)]
            for j in range(16):
                q_v[pl.ds(j * 800 + o, 16)] = (
                    (p_v[pl.ds(j * 800 + o, 16)]
                     + p_v[pl.ds((16 + j) * 800 + o, 16)]) * y
                    + t_v[pl.ds(j * 800 + o, 16)]
                )
            return carry2
        lax.fori_loop(0, 50, posq, 0)

        for m0 in range(0, 32, 2):
            wa = [w2_v[pl.ds((j * 32 + m0) * 16, 16)] for j in range(16)]
            wb = [w2_v[pl.ds((j * 32 + m0 + 1) * 16, 16)] for j in range(16)]
            ba = b2_v[pl.ds(m0 * 16, 16)]
            bb = b2_v[pl.ds((m0 + 1) * 16, 16)]
            zero = jnp.full((16,), 0.0, _f32)

            def posh(p, carry2, wa=wa, wb=wb, ba=ba, bb=bb, m0=m0, zero=zero):
                o = p * 16
                acca = ba
                accb = bb
                for j in range(16):
                    qv = q_v[pl.ds(j * 800 + o, 16)]
                    acca = acca + qv * wa[j]
                    accb = accb + qv * wb[j]
                h_v[pl.ds(m0 * 800 + o, 16)] = jnp.maximum(acca, zero)
                h_v[pl.ds((m0 + 1) * 800 + o, 16)] = jnp.maximum(accb, zero)
                return carry2
            lax.fori_loop(0, 50, posh, 0)

        for t in range(2):
            wt = [w3_v[pl.ds((m * 2 + t) * 16, 16)] for m in range(32)]

            def posg(p, carry2, wt=wt, t=t):
                o = p * 16
                acc = jnp.full((16,), 0.0, _f32)
                for m in range(32):
                    acc = acc + h_v[pl.ds(m * 800 + o, 16)] * wt[m]
                u_v[pl.ds(t * 800 + o, 16)] = acc * di_v[pl.ds(o, 16)]
                tv_v[pl.ds(t * 800 + o, 16)] = acc * di2_v[pl.ds(o, 16)]
                return carry2
            lax.fori_loop(0, 50, posg, 0)

        for t in range(2):
            pltpu.sync_copy(u_v.at[pl.ds(t * 800, 800)],
                            u3_h.at[pl.ds(_m8(t * NPAD + pb), 800)])
            pltpu.sync_copy(tv_v.at[pl.ds(t * 800, 800)],
                            t3_h.at[pl.ds(_m8(t * NPAD + pb), 800)])
        return carry
    lax.fori_loop(0, 4, piece, 0)


@functools.partial(
    pl.kernel,
    out_type=jax.ShapeDtypeStruct((2 * NPAD,), _f32),
    mesh=_mesh(),
    compiler_params=pltpu.CompilerParams(needs_layout_passes=False),
    scratch_types=[
        pltpu.VMEM((4 * 3200,), _f32),
        pltpu.VMEM((2 * 3200,), _f32),
        pltpu.VMEM((3200,), _f32),
        pltpu.VMEM((2 * 16,), _f32),
        pltpu.VMEM((2 * 3200,), _f32),
    ],
)
def _final(p_h, t_h, di_h, b_h, o_h, p_v, t_v, di_v, b_v, o_v):
    nb = _worker_slice()
    pltpu.sync_copy(b_h, b_v)
    for t in range(2):
        pltpu.sync_copy(p_h.at[pl.ds(_m8(t * NPAD + nb), 3200)],
                        p_v.at[pl.ds(t * 3200, 3200)])
        pltpu.sync_copy(p_h.at[pl.ds(_m8((2 + t) * NPAD + nb), 3200)],
                        p_v.at[pl.ds((2 + t) * 3200, 3200)])
        pltpu.sync_copy(t_h.at[pl.ds(_m8(t * NPAD + nb), 3200)],
                        t_v.at[pl.ds(t * 3200, 3200)])
    pltpu.sync_copy(di_h.at[pl.ds(nb, 3200)], di_v)

    def pos(p, carry):
        o = p * 16
        y = di_v[pl.ds(o, 16)]
        for t in range(2):
            o_v[pl.ds(t * 3200 + o, 16)] = (
                (p_v[pl.ds(t * 3200 + o, 16)]
                 + p_v[pl.ds((2 + t) * 3200 + o, 16)]) * y
                + t_v[pl.ds(t * 3200 + o, 16)] + b_v[pl.ds(t * 16, 16)]
            )
        return carry
    lax.fori_loop(0, 200, pos, 0)

    for t in range(2):
        pltpu.sync_copy(o_v.at[pl.ds(t * 3200, 3200)],
                        o_h.at[pl.ds(_m8(t * NPAD + nb), 3200)])


_prop_deg = _make_prop(1, gather=False)
_prop_3 = _make_prop(3, gather=True)
_prop_16 = _make_prop(16, gather=True)
_prop_2 = _make_prop(2, gather=True)


def _bcast_rows(w):
    return jnp.tile(w.reshape(-1, 1), (1, 16)).astype(_f32).reshape(-1)


def kernel(x, edge_index, W1, b1, W2, b2, W3, b3):
    src = edge_index[0].astype(_i32)
    dst = edge_index[1].astype(_i32)
    pad_i = jnp.arange(EP - E, dtype=_i32)
    pad_node = N + (pad_i % 1024)
    src_p = jnp.concatenate([src, pad_node])
    dst_p = jnp.concatenate([dst, pad_node]).reshape(EP // 128, 128)
    xt = jnp.zeros((3, NPAD), _f32).at[:, :N].set(x.T).reshape(-1)
    z = jnp.zeros((NPAD,), _f32)

    dp = _prop_deg(jnp.zeros((NPAD,), _f32), src_p, dst_p, z)
    u1, t1, dinv, dinv2 = _prep0(dp, xt)
    p1 = _prop_3(u1, src_p, dst_p, z)
    u2, t2 = _mix1(p1, t1, dinv, dinv2, _bcast_rows(W1), _bcast_rows(b1))
    p2 = _prop_16(u2, src_p, dst_p, z)
    u3, t3 = _mix2(p2, t2, dinv, dinv2, _bcast_rows(W2), _bcast_rows(b2),
                   _bcast_rows(W3))
    p3 = _prop_2(u3, src_p, dst_p, z)
    outc = _final(p3, t3, dinv, _bcast_rows(b3))
    return outc.reshape(2, NPAD)[:, :N].T


# speedup vs baseline: 1.0232x; 1.0232x over previous
import functools

import jax
import jax.numpy as jnp
from jax import lax
from jax.experimental import pallas as pl
from jax.experimental.pallas import tpu as pltpu
from jax.experimental.pallas import tpu_sc as plsc

N = 100000
E = 6400000
NPAD = 102400
EP = 6553600
EW = EP // 32
K = 2048
NCHUNK = EW // K
KR = K // 128
GU = 16
NSL = NPAD // 16

_f32 = jnp.float32
_i32 = jnp.int32


def _m8(v):
    return pl.multiple_of(v, 8)


def _mesh():
    return plsc.VectorSubcoreMesh(core_axis_name="c", subcore_axis_name="s")


def _rsqrt16(x):
    i = lax.bitcast_convert_type(x, _i32)
    i = jnp.full((16,), 0x5F3759DF, _i32) - lax.shift_right_logical(i, 1)
    y = lax.bitcast_convert_type(i, _f32)
    half = x * jnp.full((16,), 0.5, _f32)
    c15 = jnp.full((16,), 1.5, _f32)
    for _ in range(3):
        y = y * (c15 - half * y * y)
    return y


def _make_prop(D, gather):
    scratch = [
        pltpu.VMEM((NPAD,), _f32),
        pltpu.VMEM((K,), _i32),
        pltpu.VMEM((K,), _i32),
        pltpu.VMEM((KR, 128), _i32),
        pltpu.VMEM((KR, 128), _i32),
        pltpu.VMEM((K,), _f32),
        pltpu.VMEM((K,), _f32),
        pltpu.VMEM_SHARED((NPAD,), _f32),
        pltpu.SemaphoreType.DMA,
        pltpu.SemaphoreType.DMA,
    ]

    @functools.partial(
        pl.kernel,
        out_type=jax.ShapeDtypeStruct((2 * D * NPAD,), _f32),
        mesh=_mesh(),
        compiler_params=pltpu.CompilerParams(needs_layout_passes=False),
        scratch_types=scratch,
    )
    def prop(u_h, src_h, dst_h, z_h, out_h, tab_v, src_v0, src_v1,
             dst_v0, dst_v1, val_v0, val_v1, acc, sem, sem2):
        c = lax.axis_index("c")
        s = lax.axis_index("s")
        w = c * 16 + s
        ebase = _m8(w * EW)
        rbase = _m8(w * (EW // 128))
        src_b = (src_v0, src_v1)
        dst_b = (dst_v0, dst_v1)
        val_b = (val_v0, val_v1)

        if not gather:
            def fill(i, carry):
                val_v0[pl.ds(i * 16, 16)] = jnp.full((16,), 1.0, _f32)
                val_v1[pl.ds(i * 16, 16)] = jnp.full((16,), 1.0, _f32)
                return carry
            lax.fori_loop(0, K // 16, fill, 0)

        def start_idx_load(i, b):
            pltpu.async_copy(dst_h.at[pl.ds(_m8(rbase + i * KR), KR)],
                             dst_b[b], sem2)
            if gather:
                pltpu.async_copy(src_h.at[pl.ds(_m8(ebase + i * K), K)],
                                 src_b[b], sem2)

        def wait_idx_load(b):
            pltpu.make_async_copy(dst_h.at[pl.ds(0, KR)], dst_b[b], sem2).wait()
            if gather:
                pltpu.make_async_copy(src_h.at[pl.ds(0, K)], src_b[b],
                                      sem2).wait()

        def gather_chunk(b):
            if gather:
                def gat(k, carry3):
                    kb = k * (16 * GU)
                    for g in range(GU):
                        idx = src_b[b][pl.ds(kb + g * 16, 16)]
                        val_b[b][pl.ds(kb + g * 16, 16)] = (
                            plsc.load_gather(tab_v, [idx]))
                    return carry3
                lax.fori_loop(0, K // (16 * GU), gat, 0)

        def fire_chunk(b):
            for r in range(KR):
                pltpu.async_copy(val_b[b].at[pl.ds(r * 128, 128)],
                                 acc.at[dst_b[b].at[r]], sem, add=True)

        def feature(j, carry):
            sl = _m8(s * NSL)
            pltpu.sync_copy(z_h.at[pl.ds(sl, NSL)], acc.at[pl.ds(sl, NSL)])
            if gather:
                pltpu.sync_copy(u_h.at[pl.ds(_m8(j * NPAD), NPAD)], tab_v)
            plsc.subcore_barrier()

            start_idx_load(0, 0)

            def chunk2(ih, carry2):
                for b in range(2):
                    i = ih * 2 + b
                    wait_idx_load(b)
                    gather_chunk(b)

                    @pl.when(i > 0)
                    def _():
                        pltpu.make_async_copy(z_h.at[pl.ds(0, K)],
                                              val_b[1 - b], sem).wait()
                    fire_chunk(b)

                    @pl.when(i + 1 < NCHUNK)
                    def _():
                        start_idx_load(i + 1, 1 - b)
                return carry2
            lax.fori_loop(0, NCHUNK // 2, chunk2, 0)
            pltpu.make_async_copy(z_h.at[pl.ds(0, K)],
                                  val_b[(NCHUNK - 1) % 2], sem).wait()

            plsc.subcore_barrier()
            pltpu.sync_copy(acc.at[pl.ds(sl, NSL)],
                            out_h.at[pl.ds(_m8((c * D + j) * NPAD + sl), NSL)])
            plsc.subcore_barrier()
            return carry
        lax.fori_loop(0, D, feature, 0)

    return prop


def _worker_slice():
    c = lax.axis_index("c")
    s = lax.axis_index("s")
    return _m8((c * 16 + s) * 3200)


@functools.partial(
    pl.kernel,
    out_type=(
        jax.ShapeDtypeStruct((3 * NPAD,), _f32),
        jax.ShapeDtypeStruct((3 * NPAD,), _f32),
        jax.ShapeDtypeStruct((NPAD,), _f32),
        jax.ShapeDtypeStruct((NPAD,), _f32),
    ),
    mesh=_mesh(),
    compiler_params=pltpu.CompilerParams(needs_layout_passes=False),
    scratch_types=[
        pltpu.VMEM((2 * 3200,), _f32),
        pltpu.VMEM((3 * 3200,), _f32),
        pltpu.VMEM((3 * 3200,), _f32),
        pltpu.VMEM((3 * 3200,), _f32),
        pltpu.VMEM((3200,), _f32),
        pltpu.VMEM((3200,), _f32),
    ],
)
def _prep0(dp_h, xt_h, u1_h, t1_h, di_h, di2_h,
           dp_v, x_v, u_v, t_v, di_v, di2_v):
    nb = _worker_slice()
    for k in range(2):
        pltpu.sync_copy(dp_h.at[pl.ds(_m8(k * NPAD + nb), 3200)],
                        dp_v.at[pl.ds(k * 3200, 3200)])
    for k in range(3):
        pltpu.sync_copy(xt_h.at[pl.ds(_m8(k * NPAD + nb), 3200)],
                        x_v.at[pl.ds(k * 3200, 3200)])

    one = jnp.full((16,), 1.0, _f32)

    def pos(p, carry):
        o = p * 16
        d = dp_v[pl.ds(o, 16)] + dp_v[pl.ds(3200 + o, 16)] + one
        y = _rsqrt16(d)
        y2 = y * y
        di_v[pl.ds(o, 16)] = y
        di2_v[pl.ds(o, 16)] = y2
        for k in range(3):
            xv = x_v[pl.ds(k * 3200 + o, 16)]
            u_v[pl.ds(k * 3200 + o, 16)] = xv * y
            t_v[pl.ds(k * 3200 + o, 16)] = xv * y2
        return carry
    lax.fori_loop(0, 200, pos, 0)

    for k in range(3):
        pltpu.sync_copy(u_v.at[pl.ds(k * 3200, 3200)],
                        u1_h.at[pl.ds(_m8(k * NPAD + nb), 3200)])
        pltpu.sync_copy(t_v.at[pl.ds(k * 3200, 3200)],
                        t1_h.at[pl.ds(_m8(k * NPAD + nb), 3200)])
    pltpu.sync_copy(di_v, di_h.at[pl.ds(nb, 3200)])
    pltpu.sync_copy(di2_v, di2_h.at[pl.ds(nb, 3200)])


@functools.partial(
    pl.kernel,
    out_type=(
        jax.ShapeDtypeStruct((16 * NPAD,), _f32),
        jax.ShapeDtypeStruct((16 * NPAD,), _f32),
    ),
    mesh=_mesh(),
    compiler_params=pltpu.CompilerParams(needs_layout_passes=False),
    scratch_types=[
        pltpu.VMEM((6 * 1600,), _f32),
        pltpu.VMEM((3 * 1600,), _f32),
        pltpu.VMEM((1600,), _f32),
        pltpu.VMEM((1600,), _f32),
        pltpu.VMEM((48 * 16,), _f32),
        pltpu.VMEM((16 * 16,), _f32),
        pltpu.VMEM((16 * 1600,), _f32),
        pltpu.VMEM((16 * 1600,), _f32),
    ],
)
def _mix1(p_h, t_h, di_h, di2_h, w_h, b_h, u2_h, t2_h,
          p_v, t_v, di_v, di2_v, w_v, b_v, u_v, tv_v):
    nb = _worker_slice()
    pltpu.sync_copy(w_h, w_v)
    pltpu.sync_copy(b_h, b_v)

    def piece(pc, carry):
        pb = _m8(nb + pc * 1600)
        for k in range(3):
            pltpu.sync_copy(p_h.at[pl.ds(_m8(k * NPAD + pb), 1600)],
                            p_v.at[pl.ds(k * 1600, 1600)])
            pltpu.sync_copy(p_h.at[pl.ds(_m8((3 + k) * NPAD + pb), 1600)],
                            p_v.at[pl.ds((3 + k) * 1600, 1600)])
            pltpu.sync_copy(t_h.at[pl.ds(_m8(k * NPAD + pb), 1600)],
                            t_v.at[pl.ds(k * 1600, 1600)])
        pltpu.sync_copy(di_h.at[pl.ds(pb, 1600)], di_v)
        pltpu.sync_copy(di2_h.at[pl.ds(pb, 1600)], di2_v)

        def pos(p, carry2):
            o = p * 16
            y = di_v[pl.ds(o, 16)]
            y2 = di2_v[pl.ds(o, 16)]
            zero = jnp.full((16,), 0.0, _f32)
            pk = []
            for k in range(3):
                pk.append(
                    (p_v[pl.ds(k * 1600 + o, 16)]
                     + p_v[pl.ds((3 + k) * 1600 + o, 16)]) * y
                    + t_v[pl.ds(k * 1600 + o, 16)])
            for j in range(16):
                acc = b_v[pl.ds(j * 16, 16)]
                for k in range(3):
                    acc = acc + pk[k] * w_v[pl.ds((k * 16 + j) * 16, 16)]
                h = jnp.maximum(acc, zero)
                u_v[pl.ds(j * 1600 + o, 16)] = h * y
                tv_v[pl.ds(j * 1600 + o, 16)] = h * y2
            return carry2
        lax.fori_loop(0, 100, pos, 0)

        for j in range(16):
            pltpu.sync_copy(u_v.at[pl.ds(j * 1600, 1600)],
                            u2_h.at[pl.ds(_m8(j * NPAD + pb), 1600)])
            pltpu.sync_copy(tv_v.at[pl.ds(j * 1600, 1600)],
                            t2_h.at[pl.ds(_m8(j * NPAD + pb), 1600)])
        return carry
    lax.fori_loop(0, 2, piece, 0)


@functools.partial(
    pl.kernel,
    out_type=(
        jax.ShapeDtypeStruct((2 * NPAD,), _f32),
        jax.ShapeDtypeStruct((2 * NPAD,), _f32),
    ),
    mesh=_mesh(),
    compiler_params=pltpu.CompilerParams(needs_layout_passes=False),
    scratch_types=[
        pltpu.VMEM((32 * 800,), _f32),
        pltpu.VMEM((16 * 800,), _f32),
        pltpu.VMEM((800,), _f32),
        pltpu.VMEM((800,), _f32),
        pltpu.VMEM((512 * 16,), _f32),
        pltpu.VMEM((32 * 16,), _f32),
        pltpu.VMEM((64 * 16,), _f32),
        pltpu.VMEM((16 * 800,), _f32),
        pltpu.VMEM((32 * 800,), _f32),
        pltpu.VMEM((2 * 800,), _f32),
        pltpu.VMEM((2 * 800,), _f32),
    ],
)
def _mix2(p_h, t_h, di_h, di2_h, w2_h, b2_h, w3_h, u3_h, t3_h,
          p_v, t_v, di_v, di2_v, w2_v, b2_v, w3_v, q_v, h_v, u_v, tv_v):
    nb = _worker_slice()
    pltpu.sync_copy(w2_h, w2_v)
    pltpu.sync_copy(b2_h, b2_v)
    pltpu.sync_copy(w3_h, w3_v)

    def piece(pc, carry):
        pb = _m8(nb + pc * 800)
        for j in range(16):
            pltpu.sync_copy(p_h.at[pl.ds(_m8(j * NPAD + pb), 800)],
                            p_v.at[pl.ds(j * 800, 800)])
            pltpu.sync_copy(p_h.at[pl.ds(_m8((16 + j) * NPAD + pb), 800)],
                            p_v.at[pl.ds((16 + j) * 800, 800)])
            pltpu.sync_copy(t_h.at[pl.ds(_m8(j * NPAD + pb), 800)],
                            t_v.at[pl.ds(j * 800, 800)])
        pltpu.sync_copy(di_h.at[pl.ds(pb, 800)], di_v)
        pltpu.sync_copy(di2_h.at[pl.ds(pb, 800)], di2_v)

        def posq(p, carry2):
            o = p * 16
            y = di_v[pl.ds(o, 16)]
            for j in range(16):
                q_v[pl.ds(j * 800 + o, 16)] = (
                    (p_v[pl.ds(j * 800 + o, 16)]
                     + p_v[pl.ds((16 + j) * 800 + o, 16)]) * y
                    + t_v[pl.ds(j * 800 + o, 16)]
                )
            return carry2
        lax.fori_loop(0, 50, posq, 0)

        for m0 in range(0, 32, 2):
            wa = [w2_v[pl.ds((j * 32 + m0) * 16, 16)] for j in range(16)]
            wb = [w2_v[pl.ds((j * 32 + m0 + 1) * 16, 16)] for j in range(16)]
            ba = b2_v[pl.ds(m0 * 16, 16)]
            bb = b2_v[pl.ds((m0 + 1) * 16, 16)]
            zero = jnp.full((16,), 0.0, _f32)

            def posh(p, carry2, wa=wa, wb=wb, ba=ba, bb=bb, m0=m0, zero=zero):
                o = p * 16
                acca = ba
                accb = bb
                for j in range(16):
                    qv = q_v[pl.ds(j * 800 + o, 16)]
                    acca = acca + qv * wa[j]
                    accb = accb + qv * wb[j]
                h_v[pl.ds(m0 * 800 + o, 16)] = jnp.maximum(acca, zero)
                h_v[pl.ds((m0 + 1) * 800 + o, 16)] = jnp.maximum(accb, zero)
                return carry2
            lax.fori_loop(0, 50, posh, 0)

        for t in range(2):
            wt = [w3_v[pl.ds((m * 2 + t) * 16, 16)] for m in range(32)]

            def posg(p, carry2, wt=wt, t=t):
                o = p * 16
                acc = jnp.full((16,), 0.0, _f32)
                for m in range(32):
                    acc = acc + h_v[pl.ds(m * 800 + o, 16)] * wt[m]
                u_v[pl.ds(t * 800 + o, 16)] = acc * di_v[pl.ds(o, 16)]
                tv_v[pl.ds(t * 800 + o, 16)] = acc * di2_v[pl.ds(o, 16)]
                return carry2
            lax.fori_loop(0, 50, posg, 0)

        for t in range(2):
            pltpu.sync_copy(u_v.at[pl.ds(t * 800, 800)],
                            u3_h.at[pl.ds(_m8(t * NPAD + pb), 800)])
            pltpu.sync_copy(tv_v.at[pl.ds(t * 800, 800)],
                            t3_h.at[pl.ds(_m8(t * NPAD + pb), 800)])
        return carry
    lax.fori_loop(0, 4, piece, 0)


@functools.partial(
    pl.kernel,
    out_type=jax.ShapeDtypeStruct((2 * NPAD,), _f32),
    mesh=_mesh(),
    compiler_params=pltpu.CompilerParams(needs_layout_passes=False),
    scratch_types=[
        pltpu.VMEM((4 * 3200,), _f32),
        pltpu.VMEM((2 * 3200,), _f32),
        pltpu.VMEM((3200,), _f32),
        pltpu.VMEM((2 * 16,), _f32),
        pltpu.VMEM((2 * 3200,), _f32),
    ],
)
def _final(p_h, t_h, di_h, b_h, o_h, p_v, t_v, di_v, b_v, o_v):
    nb = _worker_slice()
    pltpu.sync_copy(b_h, b_v)
    for t in range(2):
        pltpu.sync_copy(p_h.at[pl.ds(_m8(t * NPAD + nb), 3200)],
                        p_v.at[pl.ds(t * 3200, 3200)])
        pltpu.sync_copy(p_h.at[pl.ds(_m8((2 + t) * NPAD + nb), 3200)],
                        p_v.at[pl.ds((2 + t) * 3200, 3200)])
        pltpu.sync_copy(t_h.at[pl.ds(_m8(t * NPAD + nb), 3200)],
                        t_v.at[pl.ds(t * 3200, 3200)])
    pltpu.sync_copy(di_h.at[pl.ds(nb, 3200)], di_v)

    def pos(p, carry):
        o = p * 16
        y = di_v[pl.ds(o, 16)]
        for t in range(2):
            o_v[pl.ds(t * 3200 + o, 16)] = (
                (p_v[pl.ds(t * 3200 + o, 16)]
                 + p_v[pl.ds((2 + t) * 3200 + o, 16)]) * y
                + t_v[pl.ds(t * 3200 + o, 16)] + b_v[pl.ds(t * 16, 16)]
            )
        return carry
    lax.fori_loop(0, 200, pos, 0)

    for t in range(2):
        pltpu.sync_copy(o_v.at[pl.ds(t * 3200, 3200)],
                        o_h.at[pl.ds(_m8(t * NPAD + nb), 3200)])


_prop_deg = _make_prop(1, gather=False)
_prop_3 = _make_prop(3, gather=True)
_prop_16 = _make_prop(16, gather=True)
_prop_2 = _make_prop(2, gather=True)


def _bcast_rows(w):
    return jnp.tile(w.reshape(-1, 1), (1, 16)).astype(_f32).reshape(-1)


def kernel(x, edge_index, W1, b1, W2, b2, W3, b3):
    src = edge_index[0].astype(_i32)
    dst = edge_index[1].astype(_i32)
    pad_i = jnp.arange(EP - E, dtype=_i32)
    pad_node = N + (pad_i % 1024)
    src_p = jnp.concatenate([src, pad_node])
    dst_p = jnp.concatenate([dst, pad_node]).reshape(EP // 128, 128)
    xt = jnp.zeros((3, NPAD), _f32).at[:, :N].set(x.T).reshape(-1)
    z = jnp.zeros((NPAD,), _f32)

    dp = _prop_deg(jnp.zeros((NPAD,), _f32), src_p, dst_p, z)
    u1, t1, dinv, dinv2 = _prep0(dp, xt)
    p1 = _prop_3(u1, src_p, dst_p, z)
    u2, t2 = _mix1(p1, t1, dinv, dinv2, _bcast_rows(W1), _bcast_rows(b1))
    p2 = _prop_16(u2, src_p, dst_p, z)
    u3, t3 = _mix2(p2, t2, dinv, dinv2, _bcast_rows(W2), _bcast_rows(b2),
                   _bcast_rows(W3))
    p3 = _prop_2(u3, src_p, dst_p, z)
    outc = _final(p3, t3, dinv, _bcast_rows(b3))
    return outc.reshape(2, NPAD)[:, :N].T
